# baseline (speedup 1.0000x reference)
import functools

import jax
import jax.numpy as jnp
from jax import lax
from jax.experimental import pallas as pl
from jax.experimental.pallas import tpu as pltpu
from jax.experimental.pallas import tpu_sc as plsc

B, N, E = 1024, 14, 64
F_IN, HID, HEADS = 128, 256, 8
S = 14
G = 16
R = G * S
LH = HEADS * S
VSUB = 4
STEPS = B // (G * VSUB)
NSQ = S * S


_NC, _NS = 2, 16
_NW = _NC * _NS
_GPW = B // _NW
_CH = 32
_NCHUNK = _GPW // _CH


def _sc_counts_body(el_hbm, out_hbm, el_v, cnt_v):
    wid = lax.axis_index("s") * _NC + lax.axis_index("c")
    ones = jnp.ones((16,), jnp.float32)
    zeros = jnp.zeros((16,), jnp.float32)
    io = lax.iota(jnp.int32, 16)
    selfmask = io < N
    ewords = _CH * E * 2
    cwords = _CH * NSQ
    for c in range(_NCHUNK):
        base = wid * _GPW + c * _CH
        pltpu.sync_copy(el_hbm.at[pl.ds(base * E * 2, ewords)], el_v)
        for q in range(cwords // 16):
            cnt_v[pl.ds(q * 16, 16)] = zeros
        for g in range(_CH):
            goff = g * NSQ
            for j in range(E // 16):
                sidx = g * (E * 2) + j * 32 + io * 2
                s = plsc.load_gather(el_v, [sidx])
                d = plsc.load_gather(el_v, [sidx + 1])
                plsc.addupdate_scatter(cnt_v, [goff + d * S + s], ones)
            plsc.addupdate_scatter(cnt_v, [goff + io * (S + 1)], ones,
                                   mask=selfmask)
        pltpu.sync_copy(cnt_v, out_hbm.at[pl.ds(base * NSQ, cwords)])


def _sc_counts(el_flat):
    mesh = plsc.VectorSubcoreMesh(core_axis_name="c", subcore_axis_name="s")
    fn = functools.partial(
        pl.kernel,
        mesh=mesh,
        out_type=jax.ShapeDtypeStruct((B * NSQ,), jnp.float32),
        scratch_types=[
            pltpu.VMEM((_CH * E * 2,), jnp.int32),
            pltpu.VMEM((_CH * NSQ,), jnp.float32),
        ],
        compiler_params=pltpu.CompilerParams(needs_layout_passes=False),
    )(_sc_counts_body)
    return fn(el_flat)


def _dot(a, b):
    return jnp.dot(a, b, preferred_element_type=jnp.float32)


def _tc_body(x_ref, cnt_ref, w1_ref, aw1_ref, b1_ref, w2_ref, aw2_ref,
             b2_ref, wl_ref, bl_ref, wpt_ref, bp_ref,
             bsum_ref, permb_ref, msrc_ref, t16b_ref, tileh_ref, cexph_ref,
             sumh_ref, out_ref):
    bf16 = jnp.bfloat16
    bsum = bsum_ref[...]
    msrc = msrc_ref[...]
    tileh = tileh_ref[...]

    def gat(cmask, cl, xin, xf, w, aw_ref, bvec):
        sa = _dot(xf, aw_ref[...])
        asrc = sa[:, :HEADS]
        adst = sa[:, HEADS:]
        dstp = _dot(adst, tileh)
        srcp = _dot(bsum, _dot(asrc, tileh) * msrc)
        al = srcp + dstp
        al = jnp.where(al >= 0.0, al, 0.2 * al)
        mrow = jnp.max(jnp.where(cmask, al, -1e30), axis=1, keepdims=True)
        ee = jnp.where(cmask, cl * jnp.exp(al - mrow), 0.0)
        ssum = _dot(_dot(ee, sumh_ref[...]), tileh)
        att = (ee / (ssum + 1e-16)).astype(bf16)
        h = _dot(xin, w)
        hb = h.astype(bf16)
        t16b = t16b_ref[...]
        out = None
        for hh in range(HEADS):
            ah = att[:, hh * S:(hh + 1) * S]
            bd = (_dot(ah, t16b) * bsum).astype(bf16)
            part = _dot(bd, hb[:, hh * HID:(hh + 1) * HID])
            out = part if out is None else out + part
        return (out + bvec[...]).astype(bf16)

    wl = wl_ref[...]
    for v in range(VSUB):
        x = x_ref[v * R:(v + 1) * R, :]
        cnt = cnt_ref[v * R:(v + 1) * R, :]
        cl = _dot(cnt, cexph_ref[...])
        cmask = cl > 0.0
        h1 = jnp.maximum(
            gat(cmask, cl, x.astype(bf16), x, w1_ref[...], aw1_ref, b1_ref),
            0.0)
        h2 = gat(cmask, cl, h1, h1.astype(jnp.float32), w2_ref[...], aw2_ref,
                 b2_ref)
        hp = _dot(permb_ref[...], h2).astype(bf16)
        zacc = None
        for d in range(N):
            part = _dot(hp[d * G:(d + 1) * G, :],
                        wl[d * HID:(d + 1) * HID, :])
            zacc = part if zacc is None else zacc + part
        z = zacc + bl_ref[...]
        logit = jnp.sum(z * wpt_ref[...], axis=1, keepdims=True) + bp_ref[...]
        out_ref[v * G:(v + 1) * G, :] = 1.0 / (1.0 + jnp.exp(-logit))


def _full(shape):
    return pl.BlockSpec(shape, lambda i: tuple(0 for _ in shape))


def kernel(feature, edge_list, W1, a_src1, a_dst1, b1, W2, a_src2, a_dst2, b2,
           Wl, bl, Wp, bp):
    f32 = jnp.float32
    bf16 = jnp.bfloat16
    el_flat = edge_list.astype(jnp.int32).reshape(B * E * 2)
    counts = _sc_counts(el_flat)
    cnt_rows = counts.reshape(B * S, S)

    xp = feature.reshape(B * S, F_IN)

    def mk_a(a):
        return (a[:, :, None] * jnp.eye(HEADS, dtype=f32)[:, None, :]) \
            .reshape(HEADS * HID, HEADS)

    aw1 = jnp.dot(W1, jnp.concatenate([mk_a(a_src1), mk_a(a_dst1)], axis=1))
    aw2 = jnp.dot(W2, jnp.concatenate([mk_a(a_src2), mk_a(a_dst2)], axis=1))

    rr = jnp.arange(R)[:, None]
    cc = jnp.arange(R)[None, :]
    bsum = (rr // S == cc // S).astype(f32)
    permb = ((rr % G == cc // S) & (rr // G == cc % S)).astype(bf16)
    ll = jnp.arange(LH)[None, :]
    msrc = (rr % S == ll % S).astype(f32)
    t16b = (jnp.arange(R)[None, :] % S ==
            jnp.arange(S)[:, None]).astype(bf16) / HEADS
    tileh = (ll // S == jnp.arange(HEADS)[:, None]).astype(f32)
    cexph = (ll % S == jnp.arange(S)[:, None]).astype(f32)
    sumh = (jnp.arange(LH)[:, None] // S ==
            jnp.arange(HEADS)[None, :]).astype(f32)

    grid = (STEPS,)
    out = pl.pallas_call(
        _tc_body,
        grid=grid,
        in_specs=[
            pl.BlockSpec((VSUB * R, F_IN), lambda i: (i, 0)),
            pl.BlockSpec((VSUB * R, S), lambda i: (i, 0)),
            _full((F_IN, HEADS * HID)),
            _full((F_IN, 2 * HEADS)),
            _full((1, HID)),
            _full((HID, HEADS * HID)),
            _full((HID, 2 * HEADS)),
            _full((1, HID)),
            _full((N * HID, HID // 2)),
            _full((1, HID // 2)),
            _full((1, HID // 2)),
            _full((1, 1)),
            _full((R, R)),
            _full((R, R)),
            _full((R, LH)),
            _full((S, R)),
            _full((HEADS, LH)),
            _full((S, LH)),
            _full((LH, HEADS)),
        ],
        out_specs=pl.BlockSpec((VSUB * G, 1), lambda i: (i, 0)),
        out_shape=jax.ShapeDtypeStruct((B, 1), f32),
        compiler_params=pltpu.CompilerParams(
            dimension_semantics=("arbitrary",)),
    )(xp, cnt_rows, W1.astype(bf16), aw1, b1.reshape(1, HID),
      W2.astype(bf16), aw2, b2.reshape(1, HID), Wl.astype(bf16),
      bl.reshape(1, HID // 2), Wp.reshape(1, HID // 2), bp.reshape(1, 1),
      bsum, permb, msrc, t16b, tileh, cexph, sumh)
    return out


# speedup vs baseline: 1.0060x; 1.0060x over previous
import functools

import jax
import jax.numpy as jnp
from jax import lax
from jax.experimental import pallas as pl
from jax.experimental.pallas import tpu as pltpu
from jax.experimental.pallas import tpu_sc as plsc

B, N, E = 1024, 14, 64
F_IN, HID, HEADS = 128, 256, 8
S = 14
G = 16
R = G * S
LH = HEADS * S
VSUB = 8
STEPS = B // (G * VSUB)
NSQ = S * S


_NC, _NS = 2, 16
_NW = _NC * _NS
_GPW = B // _NW
_CH = 32
_NCHUNK = _GPW // _CH


def _sc_counts_body(el_hbm, out_hbm, el_v, cnt_v):
    wid = lax.axis_index("s") * _NC + lax.axis_index("c")
    ones = jnp.ones((16,), jnp.float32)
    zeros = jnp.zeros((16,), jnp.float32)
    io = lax.iota(jnp.int32, 16)
    selfmask = io < N
    ewords = _CH * E * 2
    cwords = _CH * NSQ
    for c in range(_NCHUNK):
        base = wid * _GPW + c * _CH
        pltpu.sync_copy(el_hbm.at[pl.ds(base * E * 2, ewords)], el_v)
        for q in range(cwords // 16):
            cnt_v[pl.ds(q * 16, 16)] = zeros
        for g in range(_CH):
            goff = g * NSQ
            for j in range(E // 16):
                sidx = g * (E * 2) + j * 32 + io * 2
                s = plsc.load_gather(el_v, [sidx])
                d = plsc.load_gather(el_v, [sidx + 1])
                plsc.addupdate_scatter(cnt_v, [goff + d * S + s], ones)
            plsc.addupdate_scatter(cnt_v, [goff + io * (S + 1)], ones,
                                   mask=selfmask)
        pltpu.sync_copy(cnt_v, out_hbm.at[pl.ds(base * NSQ, cwords)])


def _sc_counts(el_flat):
    mesh = plsc.VectorSubcoreMesh(core_axis_name="c", subcore_axis_name="s")
    fn = functools.partial(
        pl.kernel,
        mesh=mesh,
        out_type=jax.ShapeDtypeStruct((B * NSQ,), jnp.float32),
        scratch_types=[
            pltpu.VMEM((_CH * E * 2,), jnp.int32),
            pltpu.VMEM((_CH * NSQ,), jnp.float32),
        ],
        compiler_params=pltpu.CompilerParams(needs_layout_passes=False),
    )(_sc_counts_body)
    return fn(el_flat)


def _dot(a, b):
    return jnp.dot(a, b, preferred_element_type=jnp.float32)


def _tc_body(x_ref, cnt_ref, w1_ref, aw1_ref, b1_ref, w2_ref, aw2_ref,
             b2_ref, wl_ref, bl_ref, wpt_ref, bp_ref,
             bsum_ref, permb_ref, msrc_ref, t16b_ref, tileh_ref, cexph_ref,
             sumh_ref, out_ref):
    bf16 = jnp.bfloat16
    bsum = bsum_ref[...]
    msrc = msrc_ref[...]
    tileh = tileh_ref[...]

    def gat(cmask, cl, xin, xf, w, aw_ref, bvec):
        sa = _dot(xf, aw_ref[...])
        asrc = sa[:, :HEADS]
        adst = sa[:, HEADS:]
        dstp = _dot(adst, tileh)
        srcp = _dot(bsum, _dot(asrc, tileh) * msrc)
        al = srcp + dstp
        al = jnp.where(al >= 0.0, al, 0.2 * al)
        mrow = jnp.max(jnp.where(cmask, al, -1e30), axis=1, keepdims=True)
        ee = jnp.where(cmask, cl * jnp.exp(al - mrow), 0.0)
        ssum = _dot(_dot(ee, sumh_ref[...]), tileh)
        att = (ee / (ssum + 1e-16)).astype(bf16)
        h = _dot(xin, w)
        hb = h.astype(bf16)
        t16b = t16b_ref[...]
        out = None
        for hh in range(HEADS):
            ah = att[:, hh * S:(hh + 1) * S]
            bd = (_dot(ah, t16b) * bsum).astype(bf16)
            part = _dot(bd, hb[:, hh * HID:(hh + 1) * HID])
            out = part if out is None else out + part
        return (out + bvec[...]).astype(bf16)

    wl = wl_ref[...]
    for v in range(VSUB):
        x = x_ref[v * R:(v + 1) * R, :]
        cnt = cnt_ref[v * R:(v + 1) * R, :]
        cl = _dot(cnt, cexph_ref[...])
        cmask = cl > 0.0
        h1 = jnp.maximum(
            gat(cmask, cl, x.astype(bf16), x, w1_ref[...], aw1_ref, b1_ref),
            0.0)
        h2 = gat(cmask, cl, h1, h1.astype(jnp.float32), w2_ref[...], aw2_ref,
                 b2_ref)
        hp = _dot(permb_ref[...], h2).astype(bf16)
        zacc = None
        for d in range(N):
            part = _dot(hp[d * G:(d + 1) * G, :],
                        wl[d * HID:(d + 1) * HID, :])
            zacc = part if zacc is None else zacc + part
        z = zacc + bl_ref[...]
        logit = jnp.sum(z * wpt_ref[...], axis=1, keepdims=True) + bp_ref[...]
        out_ref[v * G:(v + 1) * G, :] = 1.0 / (1.0 + jnp.exp(-logit))


def _full(shape):
    return pl.BlockSpec(shape, lambda i: tuple(0 for _ in shape))


def kernel(feature, edge_list, W1, a_src1, a_dst1, b1, W2, a_src2, a_dst2, b2,
           Wl, bl, Wp, bp):
    f32 = jnp.float32
    bf16 = jnp.bfloat16
    el_flat = edge_list.astype(jnp.int32).reshape(B * E * 2)
    counts = _sc_counts(el_flat)
    cnt_rows = counts.reshape(B * S, S)

    xp = feature.reshape(B * S, F_IN)

    def mk_a(a):
        return (a[:, :, None] * jnp.eye(HEADS, dtype=f32)[:, None, :]) \
            .reshape(HEADS * HID, HEADS)

    aw1 = jnp.dot(W1, jnp.concatenate([mk_a(a_src1), mk_a(a_dst1)], axis=1))
    aw2 = jnp.dot(W2, jnp.concatenate([mk_a(a_src2), mk_a(a_dst2)], axis=1))

    rr = jnp.arange(R)[:, None]
    cc = jnp.arange(R)[None, :]
    bsum = (rr // S == cc // S).astype(f32)
    permb = ((rr % G == cc // S) & (rr // G == cc % S)).astype(bf16)
    ll = jnp.arange(LH)[None, :]
    msrc = (rr % S == ll % S).astype(f32)
    t16b = (jnp.arange(R)[None, :] % S ==
            jnp.arange(S)[:, None]).astype(bf16) / HEADS
    tileh = (ll // S == jnp.arange(HEADS)[:, None]).astype(f32)
    cexph = (ll % S == jnp.arange(S)[:, None]).astype(f32)
    sumh = (jnp.arange(LH)[:, None] // S ==
            jnp.arange(HEADS)[None, :]).astype(f32)

    grid = (STEPS,)
    out = pl.pallas_call(
        _tc_body,
        grid=grid,
        in_specs=[
            pl.BlockSpec((VSUB * R, F_IN), lambda i: (i, 0)),
            pl.BlockSpec((VSUB * R, S), lambda i: (i, 0)),
            _full((F_IN, HEADS * HID)),
            _full((F_IN, 2 * HEADS)),
            _full((1, HID)),
            _full((HID, HEADS * HID)),
            _full((HID, 2 * HEADS)),
            _full((1, HID)),
            _full((N * HID, HID // 2)),
            _full((1, HID // 2)),
            _full((1, HID // 2)),
            _full((1, 1)),
            _full((R, R)),
            _full((R, R)),
            _full((R, LH)),
            _full((S, R)),
            _full((HEADS, LH)),
            _full((S, LH)),
            _full((LH, HEADS)),
        ],
        out_specs=pl.BlockSpec((VSUB * G, 1), lambda i: (i, 0)),
        out_shape=jax.ShapeDtypeStruct((B, 1), f32),
        compiler_params=pltpu.CompilerParams(
            dimension_semantics=("arbitrary",)),
    )(xp, cnt_rows, W1.astype(bf16), aw1, b1.reshape(1, HID),
      W2.astype(bf16), aw2, b2.reshape(1, HID), Wl.astype(bf16),
      bl.reshape(1, HID // 2), Wp.reshape(1, HID // 2), bp.reshape(1, 1),
      bsum, permb, msrc, t16b, tileh, cexph, sumh)
    return out
